# baseline (speedup 1.0000x reference)
import functools

import jax
import jax.numpy as jnp
from jax import lax
from jax.experimental import pallas as pl
from jax.experimental.pallas import tpu as pltpu
from jax.experimental.pallas import tpu_sc as plsc

B = 16384
F = 26
V = 100000
V2 = 100352
VB = V2 // 2
D = 32
H1, H2, H3 = 1024, 512, 256

NFA, NFB = 16, 10
GROUPS = ((0, NFA), (NFA, NFB))

NC, NS = 2, 16
NW = NC * NS
BPW = B // NW
CHUNK = 128
NJ = BPW // CHUNK
NG = CHUNK // 16
RING = 4

_SC_PARAMS = pltpu.CompilerParams(
    needs_layout_passes=False, use_tc_tiling_on_sc=False)
_sc_mesh = lambda: plsc.VectorSubcoreMesh(
    core_axis_name="c", subcore_axis_name="s")


def _proj_body(w_ref, embt_ref, lint_ref, wl_ref, ptab_ref, ltab_ref):
    w = w_ref[0]
    x = embt_ref[0]
    ptab_ref[...] = jnp.sum(x * w.reshape(D, 1), axis=0)
    ltab_ref[...] = lint_ref[0, 0] * wl_ref[0, 0, 0]


def _project(f0, nf, proj2, emb_t, lin_t, wl2):
    return pl.pallas_call(
        _proj_body,
        grid=(nf, V2 // VB),
        in_specs=[
            pl.BlockSpec((1, 1, D), lambda f, j: (f0 + f, 0, 0)),
            pl.BlockSpec((1, D, VB), lambda f, j: (f0 + f, 0, j)),
            pl.BlockSpec((1, 1, VB), lambda f, j: (f0 + f, 0, j)),
            pl.BlockSpec((1, 1, 1), lambda f, j: (f0 + f, 0, 0)),
        ],
        out_specs=[
            pl.BlockSpec((VB,), lambda f, j: (f * (V2 // VB) + j,)),
            pl.BlockSpec((VB,), lambda f, j: (f * (V2 // VB) + j,)),
        ],
        out_shape=[
            jax.ShapeDtypeStruct((nf * V2,), jnp.float32),
            jax.ShapeDtypeStruct((nf * V2,), jnp.float32),
        ],
    )(proj2, emb_t, lin_t, wl2)


def _sc_gather(nf, ids_g, ptab, ltab):
    @functools.partial(
        pl.kernel,
        out_type=[
            jax.ShapeDtypeStruct((nf * B,), jnp.float32),
            jax.ShapeDtypeStruct((B,), jnp.float32),
        ],
        mesh=_sc_mesh(),
        scratch_types=[
            pltpu.VMEM((nf, NJ, CHUNK), jnp.int32),
            pltpu.VMEM((RING, CHUNK), jnp.int32),
            pltpu.VMEM((RING, CHUNK), jnp.float32),
            pltpu.VMEM((nf, BPW), jnp.float32),
            pltpu.VMEM((BPW,), jnp.float32),
            pltpu.SemaphoreType.DMA,
        ],
        compiler_params=_SC_PARAMS,
    )
    def sc_kernel(ids_hbm, ptab_hbm, ltab_hbm, dnn_hbm, wide_hbm,
                  ids_v, idx_buf, lbuf, stage, acc_v, sem):
        wid = lax.axis_index("s") * NC + lax.axis_index("c")
        base = wid * BPW
        pltpu.sync_copy(ids_hbm.at[wid], ids_v)
        zero = jnp.zeros((16,), jnp.float32)
        for g in range(BPW // 16):
            acc_v[pl.ds(g * 16, 16)] = zero

        def descs(c, s):
            f = c // NJ
            j = lax.rem(c, NJ)
            dnn_dst = stage.at[f, pl.ds(j * CHUNK, CHUNK)]
            return (
                pltpu.make_async_copy(ptab_hbm.at[idx_buf.at[s]], dnn_dst, sem),
                pltpu.make_async_copy(ltab_hbm.at[idx_buf.at[s]], lbuf.at[s], sem),
            )

        def fire(c, s):
            f = c // NJ
            j = lax.rem(c, NJ)
            off = f * V2
            for g in range(NG):
                sl = pl.ds(g * 16, 16)
                idx_buf.at[s][sl] = ids_v.at[f, j][sl] + off
            dp, dl = descs(c, s)
            dp.start()
            dl.start()

        def consume(c, s):
            j = lax.rem(c, NJ)
            dp, dl = descs(c, s)
            dp.wait()
            dl.wait()
            rb = j * CHUNK
            for g in range(NG):
                sl = pl.ds(rb + g * 16, 16)
                acc_v[sl] = acc_v[sl] + lbuf.at[s][pl.ds(g * 16, 16)]

        for s in range(RING):
            fire(s, s)

        def body(q, carry):
            cb = RING * q
            for s in range(RING):
                c = cb + s
                consume(c, s)
                fire(c + RING, s)
            return carry

        nch = nf * NJ
        lax.fori_loop(0, nch // RING - 1, body, 0)
        for s in range(RING):
            consume(nch - RING + s, s)

        for f in range(nf):
            pltpu.sync_copy(stage.at[f], dnn_hbm.at[pl.ds(f * B + base, BPW)])
        pltpu.sync_copy(acc_v, wide_hbm.at[pl.ds(base, BPW)])

    return sc_kernel(ids_g, ptab, ltab)


def _mlp_body(xa_ref, xb_ref, wa_ref, wb_ref, w1a_ref, w1b_ref, b1_ref,
              w2_ref, b2_ref, w3_ref, b3_ref, wld_ref, bl_ref, out_ref):
    dn = lambda a, x: lax.dot_general(
        a, x, (((0,), (0,)), ((), ())), preferred_element_type=jnp.float32)
    h = jnp.maximum(
        dn(w1a_ref[...], xa_ref[...]) + dn(w1b_ref[...], xb_ref[...])
        + b1_ref[...], 0.0)
    h = jnp.maximum(dn(w2_ref[...], h) + b2_ref[...], 0.0)
    h = jnp.maximum(dn(w3_ref[...], h) + b3_ref[...], 0.0)
    out = (dn(wld_ref[...], h) + bl_ref[...]
           + wa_ref[...].reshape(1, -1) + wb_ref[...].reshape(1, -1))
    out_ref[...] = out


def _mlp(xa, xb, wa, wb, W1a, W1b, b1, W2, b2, W3, b3, Wld, bl):
    BM = 2048
    full = lambda shape: pl.BlockSpec(shape, lambda i: (0,) * len(shape))
    return pl.pallas_call(
        _mlp_body,
        grid=(B // BM,),
        in_specs=[
            pl.BlockSpec((NFA, BM), lambda i: (0, i)),
            pl.BlockSpec((NFB, BM), lambda i: (0, i)),
            pl.BlockSpec((BM,), lambda i: (i,)),
            pl.BlockSpec((BM,), lambda i: (i,)),
            full((NFA, H1)), full((NFB, H1)), full((H1, 1)),
            full((H1, H2)), full((H2, 1)),
            full((H2, H3)), full((H3, 1)),
            full((H3, 1)), full((1, 1)),
        ],
        out_specs=pl.BlockSpec((1, BM), lambda i: (0, i)),
        out_shape=jax.ShapeDtypeStruct((1, B), jnp.float32),
    )(xa, xb, wa, wb, W1a, W1b, b1, W2, b2, W3, b3, Wld, bl)


def kernel(ids, linear_weights, embed_tables, dnn_proj,
           W1, b1, W2, b2, W3, b3, Wl, bl):
    ids32 = ids.astype(jnp.int32)
    ids_blk = (ids32.T.reshape(F, NW, BPW)
               .transpose(1, 0, 2)
               .reshape(NW, F, NJ, CHUNK))
    emb_t = jnp.transpose(embed_tables, (0, 2, 1))
    lin_t = jnp.transpose(linear_weights, (0, 2, 1))
    proj2 = dnn_proj[..., 0].reshape(F, 1, D)
    wl2 = Wl[:F].reshape(F, 1, 1)

    xs, ws = [], []
    for f0, nf in GROUPS:
        ptab, ltab = _project(f0, nf, proj2, emb_t, lin_t, wl2)
        dnn, wide = _sc_gather(nf, ids_blk[:, f0:f0 + nf], ptab, ltab)
        xs.append(dnn.reshape(nf, B))
        ws.append(wide)

    out = _mlp(xs[0], xs[1], ws[0], ws[1],
               W1[:NFA], W1[NFA:], b1.reshape(H1, 1),
               W2, b2.reshape(H2, 1), W3, b3.reshape(H3, 1),
               Wl[F:], bl.reshape(1, 1))
    return out.reshape(B, 1)


# speedup vs baseline: 1.0491x; 1.0491x over previous
import functools

import jax
import jax.numpy as jnp
from jax import lax
from jax.experimental import pallas as pl
from jax.experimental.pallas import tpu as pltpu
from jax.experimental.pallas import tpu_sc as plsc

B = 16384
F = 26
V = 100000
V2 = 100352
VB = V2 // 2
D = 32
H1, H2, H3 = 1024, 512, 256

NFA, NFB = 13, 13
GROUPS = ((0, NFA), (NFA, NFB))

NC, NS = 2, 16
NW = NC * NS
BPW = B // NW
CHUNK = 128
NJ = BPW // CHUNK
NG = CHUNK // 16
RING = 13

_SC_PARAMS = pltpu.CompilerParams(
    needs_layout_passes=False, use_tc_tiling_on_sc=False)
_sc_mesh = lambda: plsc.VectorSubcoreMesh(
    core_axis_name="c", subcore_axis_name="s")


def _proj_body(w_ref, embt_ref, lint_ref, wl_ref, ptab_ref, ltab_ref):
    w = w_ref[0]
    x = embt_ref[0]
    ptab_ref[...] = jnp.sum(x * w.reshape(D, 1), axis=0)
    ltab_ref[...] = lint_ref[0, 0] * wl_ref[0, 0, 0]


def _project(f0, nf, proj2, emb_t, lin_t, wl2):
    return pl.pallas_call(
        _proj_body,
        grid=(nf, V2 // VB),
        in_specs=[
            pl.BlockSpec((1, 1, D), lambda f, j: (f0 + f, 0, 0)),
            pl.BlockSpec((1, D, VB), lambda f, j: (f0 + f, 0, j)),
            pl.BlockSpec((1, 1, VB), lambda f, j: (f0 + f, 0, j)),
            pl.BlockSpec((1, 1, 1), lambda f, j: (f0 + f, 0, 0)),
        ],
        out_specs=[
            pl.BlockSpec((VB,), lambda f, j: (f * (V2 // VB) + j,)),
            pl.BlockSpec((VB,), lambda f, j: (f * (V2 // VB) + j,)),
        ],
        out_shape=[
            jax.ShapeDtypeStruct((nf * V2,), jnp.float32),
            jax.ShapeDtypeStruct((nf * V2,), jnp.float32),
        ],
    )(proj2, emb_t, lin_t, wl2)


def _sc_gather(nf, ids_g, ptab, ltab):
    @functools.partial(
        pl.kernel,
        out_type=[
            jax.ShapeDtypeStruct((nf * B,), jnp.float32),
            jax.ShapeDtypeStruct((B,), jnp.float32),
        ],
        mesh=_sc_mesh(),
        scratch_types=[
            pltpu.VMEM((nf, NJ, CHUNK), jnp.int32),
            pltpu.VMEM((RING, CHUNK), jnp.int32),
            pltpu.VMEM((RING, CHUNK), jnp.float32),
            pltpu.VMEM((nf, BPW), jnp.float32),
            pltpu.VMEM((BPW,), jnp.float32),
            pltpu.SemaphoreType.DMA,
        ],
        compiler_params=_SC_PARAMS,
    )
    def sc_kernel(ids_hbm, ptab_hbm, ltab_hbm, dnn_hbm, wide_hbm,
                  ids_v, idx_buf, lbuf, stage, acc_v, sem):
        wid = lax.axis_index("s") * NC + lax.axis_index("c")
        base = wid * BPW
        pltpu.sync_copy(ids_hbm.at[wid], ids_v)
        zero = jnp.zeros((16,), jnp.float32)
        for g in range(BPW // 16):
            acc_v[pl.ds(g * 16, 16)] = zero

        def descs(c, s):
            f = c // NJ
            j = lax.rem(c, NJ)
            dnn_dst = stage.at[f, pl.ds(j * CHUNK, CHUNK)]
            return (
                pltpu.make_async_copy(ptab_hbm.at[idx_buf.at[s]], dnn_dst, sem),
                pltpu.make_async_copy(ltab_hbm.at[idx_buf.at[s]], lbuf.at[s], sem),
            )

        def fire(c, s):
            f = c // NJ
            j = lax.rem(c, NJ)
            off = f * V2
            for g in range(NG):
                sl = pl.ds(g * 16, 16)
                idx_buf.at[s][sl] = ids_v.at[f, j][sl] + off
            dp, dl = descs(c, s)
            dp.start()
            dl.start()

        def consume(c, s):
            j = lax.rem(c, NJ)
            dp, dl = descs(c, s)
            dp.wait()
            dl.wait()
            rb = j * CHUNK
            for g in range(NG):
                sl = pl.ds(rb + g * 16, 16)
                acc_v[sl] = acc_v[sl] + lbuf.at[s][pl.ds(g * 16, 16)]

        for s in range(RING):
            fire(s, s)

        def body(q, carry):
            cb = RING * q
            for s in range(RING):
                c = cb + s
                consume(c, s)
                fire(c + RING, s)
            return carry

        nch = nf * NJ
        lax.fori_loop(0, nch // RING - 1, body, 0)
        for s in range(RING):
            consume(nch - RING + s, s)

        for f in range(nf):
            pltpu.sync_copy(stage.at[f], dnn_hbm.at[pl.ds(f * B + base, BPW)])
        pltpu.sync_copy(acc_v, wide_hbm.at[pl.ds(base, BPW)])

    return sc_kernel(ids_g, ptab, ltab)


def _mlp_body(xa_ref, xb_ref, wa_ref, wb_ref, w1a_ref, w1b_ref, b1_ref,
              w2_ref, b2_ref, w3_ref, b3_ref, wld_ref, bl_ref, out_ref):
    dn = lambda a, x: lax.dot_general(
        a, x, (((0,), (0,)), ((), ())), preferred_element_type=jnp.float32)
    h = jnp.maximum(
        dn(w1a_ref[...], xa_ref[...]) + dn(w1b_ref[...], xb_ref[...])
        + b1_ref[...], 0.0)
    h = jnp.maximum(dn(w2_ref[...], h) + b2_ref[...], 0.0)
    h = jnp.maximum(dn(w3_ref[...], h) + b3_ref[...], 0.0)
    out = (dn(wld_ref[...], h) + bl_ref[...]
           + wa_ref[...].reshape(1, -1) + wb_ref[...].reshape(1, -1))
    out_ref[...] = out


def _mlp(xa, xb, wa, wb, W1a, W1b, b1, W2, b2, W3, b3, Wld, bl):
    BM = 4096
    full = lambda shape: pl.BlockSpec(shape, lambda i: (0,) * len(shape))
    return pl.pallas_call(
        _mlp_body,
        grid=(B // BM,),
        in_specs=[
            pl.BlockSpec((NFA, BM), lambda i: (0, i)),
            pl.BlockSpec((NFB, BM), lambda i: (0, i)),
            pl.BlockSpec((BM,), lambda i: (i,)),
            pl.BlockSpec((BM,), lambda i: (i,)),
            full((NFA, H1)), full((NFB, H1)), full((H1, 1)),
            full((H1, H2)), full((H2, 1)),
            full((H2, H3)), full((H3, 1)),
            full((H3, 1)), full((1, 1)),
        ],
        out_specs=pl.BlockSpec((1, BM), lambda i: (0, i)),
        out_shape=jax.ShapeDtypeStruct((1, B), jnp.float32),
    )(xa, xb, wa, wb, W1a, W1b, b1, W2, b2, W3, b3, Wld, bl)


def kernel(ids, linear_weights, embed_tables, dnn_proj,
           W1, b1, W2, b2, W3, b3, Wl, bl):
    ids32 = ids.astype(jnp.int32)
    ids_blk = (ids32.T.reshape(F, NW, BPW)
               .transpose(1, 0, 2)
               .reshape(NW, F, NJ, CHUNK))
    emb_t = jnp.transpose(embed_tables, (0, 2, 1))
    lin_t = jnp.transpose(linear_weights, (0, 2, 1))
    proj2 = dnn_proj[..., 0].reshape(F, 1, D)
    wl2 = Wl[:F].reshape(F, 1, 1)

    xs, ws = [], []
    for f0, nf in GROUPS:
        ptab, ltab = _project(f0, nf, proj2, emb_t, lin_t, wl2)
        dnn, wide = _sc_gather(nf, ids_blk[:, f0:f0 + nf], ptab, ltab)
        xs.append(dnn.reshape(nf, B))
        ws.append(wide)

    out = _mlp(xs[0], xs[1], ws[0], ws[1],
               W1[:NFA], W1[NFA:], b1.reshape(H1, 1),
               W2, b2.reshape(H2, 1), W3, b3.reshape(H3, 1),
               Wl[F:], bl.reshape(1, 1))
    return out.reshape(B, 1)


# speedup vs baseline: 1.0527x; 1.0034x over previous
import functools

import jax
import jax.numpy as jnp
from jax import lax
from jax.experimental import pallas as pl
from jax.experimental.pallas import tpu as pltpu
from jax.experimental.pallas import tpu_sc as plsc

B = 16384
F = 26
V = 100000
V2 = 100352
VB = V2 // 2
D = 32
H1, H2, H3 = 1024, 512, 256

NFA, NFB = 13, 13
GROUPS = ((0, NFA), (NFA, NFB))

NC, NS = 2, 16
NW = NC * NS
BPW = B // NW
CHUNK = 128
NJ = BPW // CHUNK
NG = CHUNK // 16
RING = 26

_SC_PARAMS = pltpu.CompilerParams(
    needs_layout_passes=False, use_tc_tiling_on_sc=False)
_sc_mesh = lambda: plsc.VectorSubcoreMesh(
    core_axis_name="c", subcore_axis_name="s")


def _proj_body(w_ref, embt_ref, lint_ref, wl_ref, ptab_ref, ltab_ref):
    w = w_ref[0]
    x = embt_ref[0]
    ptab_ref[...] = jnp.sum(x * w.reshape(D, 1), axis=0)
    ltab_ref[...] = lint_ref[0, 0] * wl_ref[0, 0, 0]


def _project(f0, nf, proj2, emb_t, lin_t, wl2):
    return pl.pallas_call(
        _proj_body,
        grid=(nf, V2 // VB),
        in_specs=[
            pl.BlockSpec((1, 1, D), lambda f, j: (f0 + f, 0, 0)),
            pl.BlockSpec((1, D, VB), lambda f, j: (f0 + f, 0, j)),
            pl.BlockSpec((1, 1, VB), lambda f, j: (f0 + f, 0, j)),
            pl.BlockSpec((1, 1, 1), lambda f, j: (f0 + f, 0, 0)),
        ],
        out_specs=[
            pl.BlockSpec((VB,), lambda f, j: (f * (V2 // VB) + j,)),
            pl.BlockSpec((VB,), lambda f, j: (f * (V2 // VB) + j,)),
        ],
        out_shape=[
            jax.ShapeDtypeStruct((nf * V2,), jnp.float32),
            jax.ShapeDtypeStruct((nf * V2,), jnp.float32),
        ],
    )(proj2, emb_t, lin_t, wl2)


def _sc_gather(nf, ids_g, ptab, ltab):
    @functools.partial(
        pl.kernel,
        out_type=[
            jax.ShapeDtypeStruct((nf * B,), jnp.float32),
            jax.ShapeDtypeStruct((B,), jnp.float32),
        ],
        mesh=_sc_mesh(),
        scratch_types=[
            pltpu.VMEM((nf, NJ, CHUNK), jnp.int32),
            pltpu.VMEM((RING, CHUNK), jnp.int32),
            pltpu.VMEM((RING, CHUNK), jnp.float32),
            pltpu.VMEM((nf, BPW), jnp.float32),
            pltpu.VMEM((BPW,), jnp.float32),
            pltpu.SemaphoreType.DMA,
        ],
        compiler_params=_SC_PARAMS,
    )
    def sc_kernel(ids_hbm, ptab_hbm, ltab_hbm, dnn_hbm, wide_hbm,
                  ids_v, idx_buf, lbuf, stage, acc_v, sem):
        wid = lax.axis_index("s") * NC + lax.axis_index("c")
        base = wid * BPW
        pltpu.sync_copy(ids_hbm.at[wid], ids_v)
        zero = jnp.zeros((16,), jnp.float32)
        for g in range(BPW // 16):
            acc_v[pl.ds(g * 16, 16)] = zero

        def descs(c, s):
            f = c // NJ
            j = lax.rem(c, NJ)
            dnn_dst = stage.at[f, pl.ds(j * CHUNK, CHUNK)]
            return (
                pltpu.make_async_copy(ptab_hbm.at[idx_buf.at[s]], dnn_dst, sem),
                pltpu.make_async_copy(ltab_hbm.at[idx_buf.at[s]], lbuf.at[s], sem),
            )

        def fire(c, s):
            f = c // NJ
            j = lax.rem(c, NJ)
            off = f * V2
            for g in range(NG):
                sl = pl.ds(g * 16, 16)
                idx_buf.at[s][sl] = ids_v.at[f, j][sl] + off
            dp, dl = descs(c, s)
            dp.start()
            dl.start()

        def consume(c, s):
            j = lax.rem(c, NJ)
            dp, dl = descs(c, s)
            dp.wait()
            dl.wait()
            rb = j * CHUNK
            for g in range(NG):
                sl = pl.ds(rb + g * 16, 16)
                acc_v[sl] = acc_v[sl] + lbuf.at[s][pl.ds(g * 16, 16)]

        for s in range(RING):
            fire(s, s)

        def body(q, carry):
            cb = RING * q
            for s in range(RING):
                c = cb + s
                consume(c, s)
                fire(c + RING, s)
            return carry

        nch = nf * NJ
        lax.fori_loop(0, nch // RING - 1, body, 0)
        for s in range(RING):
            consume(nch - RING + s, s)

        for f in range(nf):
            pltpu.sync_copy(stage.at[f], dnn_hbm.at[pl.ds(f * B + base, BPW)])
        pltpu.sync_copy(acc_v, wide_hbm.at[pl.ds(base, BPW)])

    return sc_kernel(ids_g, ptab, ltab)


def _mlp_body(xa_ref, xb_ref, wa_ref, wb_ref, w1a_ref, w1b_ref, b1_ref,
              w2_ref, b2_ref, w3_ref, b3_ref, wld_ref, bl_ref, out_ref):
    dn = lambda a, x: lax.dot_general(
        a, x, (((0,), (0,)), ((), ())), preferred_element_type=jnp.float32)
    h = jnp.maximum(
        dn(w1a_ref[...], xa_ref[...]) + dn(w1b_ref[...], xb_ref[...])
        + b1_ref[...], 0.0)
    h = jnp.maximum(dn(w2_ref[...], h) + b2_ref[...], 0.0)
    h = jnp.maximum(dn(w3_ref[...], h) + b3_ref[...], 0.0)
    out = (dn(wld_ref[...], h) + bl_ref[...]
           + wa_ref[...].reshape(1, -1) + wb_ref[...].reshape(1, -1))
    out_ref[...] = out


def _mlp(xa, xb, wa, wb, W1a, W1b, b1, W2, b2, W3, b3, Wld, bl):
    BM = 4096
    full = lambda shape: pl.BlockSpec(shape, lambda i: (0,) * len(shape))
    return pl.pallas_call(
        _mlp_body,
        grid=(B // BM,),
        in_specs=[
            pl.BlockSpec((NFA, BM), lambda i: (0, i)),
            pl.BlockSpec((NFB, BM), lambda i: (0, i)),
            pl.BlockSpec((BM,), lambda i: (i,)),
            pl.BlockSpec((BM,), lambda i: (i,)),
            full((NFA, H1)), full((NFB, H1)), full((H1, 1)),
            full((H1, H2)), full((H2, 1)),
            full((H2, H3)), full((H3, 1)),
            full((H3, 1)), full((1, 1)),
        ],
        out_specs=pl.BlockSpec((1, BM), lambda i: (0, i)),
        out_shape=jax.ShapeDtypeStruct((1, B), jnp.float32),
    )(xa, xb, wa, wb, W1a, W1b, b1, W2, b2, W3, b3, Wld, bl)


def kernel(ids, linear_weights, embed_tables, dnn_proj,
           W1, b1, W2, b2, W3, b3, Wl, bl):
    ids32 = ids.astype(jnp.int32)
    ids_blk = (ids32.T.reshape(F, NW, BPW)
               .transpose(1, 0, 2)
               .reshape(NW, F, NJ, CHUNK))
    emb_t = jnp.transpose(embed_tables, (0, 2, 1))
    lin_t = jnp.transpose(linear_weights, (0, 2, 1))
    proj2 = dnn_proj[..., 0].reshape(F, 1, D)
    wl2 = Wl[:F].reshape(F, 1, 1)

    xs, ws = [], []
    for f0, nf in GROUPS:
        ptab, ltab = _project(f0, nf, proj2, emb_t, lin_t, wl2)
        dnn, wide = _sc_gather(nf, ids_blk[:, f0:f0 + nf], ptab, ltab)
        xs.append(dnn.reshape(nf, B))
        ws.append(wide)

    out = _mlp(xs[0], xs[1], ws[0], ws[1],
               W1[:NFA], W1[NFA:], b1.reshape(H1, 1),
               W2, b2.reshape(H2, 1), W3, b3.reshape(H3, 1),
               Wl[F:], bl.reshape(1, 1))
    return out.reshape(B, 1)
